# baseline (speedup 1.0000x reference)
import functools

import jax
import jax.numpy as jnp
from jax import lax
from jax.experimental import pallas as pl
from jax.experimental.pallas import tpu as pltpu
from jax.experimental.pallas import tpu_sc as plsc

EMBED_DIM = 32
HPAD = 56
EPAD = 128
NUM_CORES = 2
NUM_SUBCORES = 16
NW = NUM_CORES * NUM_SUBCORES
NB = 8


@functools.lru_cache(maxsize=None)
def _make_kernel(batch: int, hist: int):
    per_w = batch // NW
    n_blocks = per_w // NB
    mesh = plsc.VectorSubcoreMesh(core_axis_name="c", subcore_axis_name="s")

    @functools.partial(
        pl.kernel,
        mesh=mesh,
        compiler_params=pltpu.CompilerParams(use_tc_tiling_on_sc=False),
        out_type=jax.ShapeDtypeStruct((batch, HPAD, EPAD), jnp.float32),
        scratch_types=[
            pltpu.VMEM((per_w, hist), jnp.int32),
            pltpu.VMEM((2, NB, hist, EMBED_DIM), jnp.float32),
            pltpu.SemaphoreType.DMA,
            pltpu.SemaphoreType.DMA,
        ],
    )
    def emb(x_hbm, table_hbm, out_hbm, idx_v, rows_v, gsem, wsem):
        wid = lax.axis_index("s") * NUM_CORES + lax.axis_index("c")
        base = wid * per_w
        pltpu.sync_copy(x_hbm.at[wid], idx_v)

        def gather(tb, s, b):
            return pltpu.make_async_copy(
                table_hbm.at[idx_v.at[tb * NB + b]],
                rows_v.at[s].at[b],
                gsem,
            )

        def write(tb, s, b):
            return pltpu.make_async_copy(
                rows_v.at[s].at[b],
                out_hbm.at[base + tb * NB + b].at[pl.ds(0, hist),
                                                  pl.ds(0, EMBED_DIM)],
                wsem,
            )

        for b in range(NB):
            gather(0, 0, b).start()

        def body(tb, carry):
            s = lax.rem(tb, 2)
            for b in range(NB):
                gather(tb, s, b).wait()
            @pl.when(tb >= 1)
            def _():
                for b in range(NB):
                    write(tb - 1, 1 - s, b).wait()
            for b in range(NB):
                write(tb, s, b).start()
            @pl.when(tb + 1 < n_blocks)
            def _():
                for b in range(NB):
                    gather(tb + 1, 1 - s, b).start()
            return carry

        lax.fori_loop(0, n_blocks, body, 0)
        for b in range(NB):
            write(n_blocks - 1, lax.rem(n_blocks - 1, 2), b).wait()

    return emb


VBLOCK = 1664


@functools.lru_cache(maxsize=None)
def _make_table_fmt(vpad: int):
    grid = vpad // VBLOCK

    def body(in_ref, out_ref):
        z3 = in_ref[...].reshape(EMBED_DIM, VBLOCK // 4, 4)
        zt = jnp.transpose(z3, (1, 2, 0))
        out_ref[...] = zt.reshape(VBLOCK * EMBED_DIM // 128, 128)

    return pl.pallas_call(
        body,
        grid=(grid,),
        in_specs=[pl.BlockSpec((EMBED_DIM, VBLOCK), lambda i: (0, i))],
        out_specs=pl.BlockSpec((VBLOCK * EMBED_DIM // 128, 128),
                               lambda i: (i, 0)),
        out_shape=jax.ShapeDtypeStruct((vpad * EMBED_DIM // 128, 128),
                                       jnp.float32),
    )


def kernel(x, table):
    b, h = x.shape
    v = table.shape[0]
    vpad = (v + 127) // 128 * 128
    xr = x.astype(jnp.int32).reshape(NW, b // NW, h)
    tpad = jnp.pad(table.T, ((0, 0), (0, vpad - v)))
    packed = _make_table_fmt(vpad)(tpad)
    outp = _make_kernel(b, h)(xr, packed.reshape(vpad, EMBED_DIM))
    return outp[:, :h, :EMBED_DIM]


# speedup vs baseline: 3.7346x; 3.7346x over previous
import functools

import jax
import jax.numpy as jnp
from jax import lax
from jax.experimental import pallas as pl
from jax.experimental.pallas import tpu as pltpu
from jax.experimental.pallas import tpu_sc as plsc

EMBED_DIM = 32
HPAD = 56
EPAD = 128
NUM_CORES = 2
NUM_SUBCORES = 16
NW = NUM_CORES * NUM_SUBCORES
NB = 8


@functools.lru_cache(maxsize=None)
def _make_kernel(batch: int, hist: int):
    per_w = batch // NW
    n_blocks = per_w // NB
    mesh = plsc.VectorSubcoreMesh(core_axis_name="c", subcore_axis_name="s")

    @functools.partial(
        pl.kernel,
        mesh=mesh,
        compiler_params=pltpu.CompilerParams(use_tc_tiling_on_sc=False),
        out_type=jax.ShapeDtypeStruct((batch, HPAD, EPAD), jnp.float32),
        scratch_types=[
            pltpu.VMEM((per_w, hist), jnp.int32),
            pltpu.VMEM((2, NB, hist, EMBED_DIM), jnp.float32),
            pltpu.SemaphoreType.DMA,
            pltpu.SemaphoreType.DMA,
        ],
    )
    def emb(x_hbm, table_hbm, out_hbm, idx_v, rows_v, gsem, wsem):
        wid = lax.axis_index("s") * NUM_CORES + lax.axis_index("c")
        base = wid * per_w
        pltpu.sync_copy(x_hbm.at[wid], idx_v)

        def gather(tb, s, b):
            return pltpu.make_async_copy(
                table_hbm.at[idx_v.at[tb * NB + b]],
                rows_v.at[s].at[b],
                gsem,
            )

        def write(tb, s, b):
            return pltpu.make_async_copy(
                rows_v.at[s].at[b],
                out_hbm.at[base + tb * NB + b].at[pl.ds(0, hist),
                                                  pl.ds(0, EMBED_DIM)],
                wsem,
            )

        for b in range(NB):
            gather(0, 0, b).start()

        def body(tb, carry):
            s = lax.rem(tb, 2)
            for b in range(NB):
                gather(tb, s, b).wait()
            @pl.when(tb >= 1)
            def _():
                for b in range(NB):
                    write(tb - 1, 1 - s, b).wait()
            for b in range(NB):
                write(tb, s, b).start()
            @pl.when(tb + 1 < n_blocks)
            def _():
                for b in range(NB):
                    gather(tb + 1, 1 - s, b).start()
            return carry

        lax.fori_loop(0, n_blocks, body, 0)
        for b in range(NB):
            write(n_blocks - 1, lax.rem(n_blocks - 1, 2), b).wait()

    return emb


def kernel(x, table):
    b, h = x.shape
    xr = x.astype(jnp.int32).reshape(NW, b // NW, h)
    outp = _make_kernel(b, h)(xr, table)
    return outp[:, :h, :EMBED_DIM]
